# baseline (speedup 1.0000x reference)
import functools

import jax
import jax.numpy as jnp
from jax import lax
from jax.experimental import pallas as pl
from jax.experimental.pallas import tpu as pltpu
from jax.experimental.pallas import tpu_sc as plsc

VOCAB = 1002
DIM = 128
BATCH = 4096
SEQ = 200
TOTAL = BATCH * SEQ
NUM_CORES = 2
NUM_SUBCORES = 16
NW = NUM_CORES * NUM_SUBCORES
PER_W = TOTAL // NW
CHUNK = 128
NCHUNK = PER_W // CHUNK
NBUF = 5
TROWS = 64
NGROUPS = NCHUNK // NBUF

_mesh = plsc.VectorSubcoreMesh(core_axis_name="c", subcore_axis_name="s")


@functools.partial(
    pl.kernel,
    mesh=_mesh,
    out_type=jax.ShapeDtypeStruct((TOTAL, DIM), jnp.float32),
    scratch_types=[
        pltpu.VMEM((PER_W,), jnp.int32),
        pltpu.VMEM((NBUF, CHUNK, DIM), jnp.float32),
        pltpu.VMEM_SHARED((VOCAB, DIM), jnp.float32),
    ]
    + [pltpu.SemaphoreType.DMA] * NBUF
    + [pltpu.SemaphoreType.DMA] * NBUF,
)
def _emb_lookup(idx_hbm, table_hbm, out_hbm, idx_v, rows_v, table_sp, *sems):
    gsem = sems[:NBUF]
    wsem = sems[NBUF:]
    sid = lax.axis_index("s")
    wid = sid * NUM_CORES + lax.axis_index("c")
    base = wid * PER_W

    @pl.when(sid < NUM_SUBCORES - 1)
    def _():
        pltpu.sync_copy(table_hbm.at[pl.ds(sid * TROWS, TROWS)],
                        table_sp.at[pl.ds(sid * TROWS, TROWS)])

    @pl.when(sid == NUM_SUBCORES - 1)
    def _():
        tail = (NUM_SUBCORES - 1) * TROWS
        pltpu.sync_copy(table_hbm.at[pl.ds(tail, VOCAB - tail)],
                        table_sp.at[pl.ds(tail, VOCAB - tail)])

    pltpu.sync_copy(idx_hbm.at[pl.ds(base, PER_W)], idx_v)
    plsc.subcore_barrier()

    def gather(g, b):
        pltpu.async_copy(table_sp.at[idx_v.at[pl.ds(g * CHUNK, CHUNK)]],
                         rows_v.at[b], gsem[b])

    def write(g, b):
        pltpu.async_copy(rows_v.at[b], out_hbm.at[pl.ds(base + g * CHUNK, CHUNK)], wsem[b])

    def wait_gather(b):
        pltpu.make_async_copy(table_sp.at[idx_v.at[pl.ds(0, CHUNK)]],
                              rows_v.at[b], gsem[b]).wait()

    def wait_write(b):
        pltpu.make_async_copy(rows_v.at[b], out_hbm.at[pl.ds(base, CHUNK)], wsem[b]).wait()

    for b in range(NBUF):
        gather(b, b)

    def group_body(grp, carry):
        g0 = grp * NBUF
        for b in range(NBUF):
            wait_gather(b)
            write(g0 + b, b)
        for b in range(NBUF):
            wait_write(b)
            gather(g0 + NBUF + b, b)
        return carry

    lax.fori_loop(0, NGROUPS - 1, group_body, 0)

    g0 = (NGROUPS - 1) * NBUF
    for b in range(NBUF):
        wait_gather(b)
        write(g0 + b, b)
    for b in range(NBUF):
        wait_write(b)


def kernel(input, table):
    idx = input.reshape(TOTAL)
    out = _emb_lookup(idx, table)
    return out.reshape(BATCH, SEQ, DIM)


# speedup vs baseline: 1.0086x; 1.0086x over previous
import functools

import jax
import jax.numpy as jnp
from jax import lax
from jax.experimental import pallas as pl
from jax.experimental.pallas import tpu as pltpu
from jax.experimental.pallas import tpu_sc as plsc

VOCAB = 1002
DIM = 128
BATCH = 4096
SEQ = 200
TOTAL = BATCH * SEQ
NUM_CORES = 2
NUM_SUBCORES = 16
NW = NUM_CORES * NUM_SUBCORES
PER_W = TOTAL // NW
CHUNK = 128
NCHUNK = PER_W // CHUNK
NBUF = 4
TROWS = 64
NGROUPS = NCHUNK // NBUF

_mesh = plsc.VectorSubcoreMesh(core_axis_name="c", subcore_axis_name="s")


@functools.partial(
    pl.kernel,
    mesh=_mesh,
    out_type=jax.ShapeDtypeStruct((TOTAL, DIM), jnp.float32),
    scratch_types=[
        pltpu.VMEM((NCHUNK, CHUNK), jnp.int32),
        pltpu.VMEM((NBUF, CHUNK, DIM), jnp.float32),
        pltpu.VMEM_SHARED((VOCAB, DIM), jnp.float32),
    ]
    + [pltpu.SemaphoreType.DMA] * NBUF
    + [pltpu.SemaphoreType.DMA] * NBUF,
)
def _emb_lookup(idx_hbm, table_hbm, out_hbm, idx_v, rows_v, table_sp, *sems):
    gsem = sems[:NBUF]
    wsem = sems[NBUF:]
    sid = lax.axis_index("s")
    wid = sid * NUM_CORES + lax.axis_index("c")
    base = wid * PER_W

    @pl.when(sid < NUM_SUBCORES - 1)
    def _():
        pltpu.sync_copy(table_hbm.at[pl.ds(sid * TROWS, TROWS)],
                        table_sp.at[pl.ds(sid * TROWS, TROWS)])

    @pl.when(sid == NUM_SUBCORES - 1)
    def _():
        tail = (NUM_SUBCORES - 1) * TROWS
        pltpu.sync_copy(table_hbm.at[pl.ds(tail, VOCAB - tail)],
                        table_sp.at[pl.ds(tail, VOCAB - tail)])

    pltpu.sync_copy(idx_hbm.at[wid], idx_v)
    plsc.subcore_barrier()

    def gather(g, b):
        pltpu.async_copy(table_sp.at[idx_v.at[g]], rows_v.at[b], gsem[b])

    def write(g, b):
        pltpu.async_copy(rows_v.at[b], out_hbm.at[pl.ds(base + g * CHUNK, CHUNK)], wsem[b])

    def wait_gather(b):
        pltpu.make_async_copy(table_sp.at[idx_v.at[0]], rows_v.at[b], gsem[b]).wait()

    def wait_write(b):
        pltpu.make_async_copy(rows_v.at[b], out_hbm.at[pl.ds(base, CHUNK)], wsem[b]).wait()

    for b in range(NBUF):
        gather(b, b)

    def group_body(grp, carry):
        g0 = grp * NBUF
        for b in range(NBUF):
            wait_gather(b)
            write(g0 + b, b)
        for b in range(NBUF):
            wait_write(b)
            gather(g0 + NBUF + b, b)
        return carry

    lax.fori_loop(0, NGROUPS - 1, group_body, 0)

    g0 = (NGROUPS - 1) * NBUF
    for b in range(NBUF):
        wait_gather(b)
        write(g0 + b, b)
    for b in range(NBUF):
        wait_write(b)


def kernel(input, table):
    idx = input.reshape(NW, NCHUNK, CHUNK)
    out = _emb_lookup(idx, table)
    return out.reshape(BATCH, SEQ, DIM)
